# baseline (speedup 1.0000x reference)
import functools

import jax
import jax.numpy as jnp
from jax import lax
from jax.experimental import pallas as pl
from jax.experimental.pallas import tpu as pltpu
from jax.experimental.pallas import tpu_sc as plsc


def _sc_gather(table, idx, B, D):
    NW = 32
    b_per_w = B // NW
    mesh = plsc.VectorSubcoreMesh(core_axis_name="c", subcore_axis_name="s")

    @functools.partial(
        pl.kernel,
        mesh=mesh,
        out_type=jax.ShapeDtypeStruct((B, D), jnp.float32),
        scratch_types=[
            pltpu.VMEM((b_per_w,), jnp.int32),
            pltpu.VMEM((b_per_w, D), jnp.float32),
            pltpu.SemaphoreType.DMA,
        ],
    )
    def gather_kernel(table_hbm, idx_hbm, out_hbm, idx_v, rows_v, sem):
        wid = lax.axis_index("s") * 2 + lax.axis_index("c")
        base = wid * b_per_w
        pltpu.sync_copy(idx_hbm.at[pl.ds(base, b_per_w)], idx_v)
        pltpu.async_copy(table_hbm.at[idx_v], rows_v, sem).wait()
        pltpu.sync_copy(rows_v, out_hbm.at[pl.ds(base, b_per_w)])

    return gather_kernel(table, idx)


def _enc_body(x_ref, w1_ref, b1_ref, w2_ref, b2_ref, z_ref):
    x = x_ref[...].astype(jnp.bfloat16)
    w1 = w1_ref[...].astype(jnp.bfloat16)
    h = jnp.dot(x, w1, preferred_element_type=jnp.float32)
    h = jnp.maximum(h + b1_ref[...], 0.0).astype(jnp.bfloat16)
    z = jnp.dot(h, w2_ref[...].astype(jnp.bfloat16),
                preferred_element_type=jnp.float32)
    z_ref[...] = z + b2_ref[...]


def _encoder(x, w1, b1, w2, b2, bm):
    B, G = x.shape
    H = w1.shape[1]
    L = w2.shape[1]
    grid = (B // bm,)
    return pl.pallas_call(
        _enc_body,
        grid=grid,
        in_specs=[
            pl.BlockSpec((bm, G), lambda i: (i, 0)),
            pl.BlockSpec((G, H), lambda i: (0, 0)),
            pl.BlockSpec((1, H), lambda i: (0, 0)),
            pl.BlockSpec((H, L), lambda i: (0, 0)),
            pl.BlockSpec((1, L), lambda i: (0, 0)),
        ],
        out_specs=pl.BlockSpec((bm, L), lambda i: (i, 0)),
        out_shape=jax.ShapeDtypeStruct((B, L), jnp.float32),
    )(x, w1, b1, w2, b2)


def _dec_body(z_ref, s_ref, w3_ref, b3_ref, w4_ref, b4_ref, o_ref):
    zp = (z_ref[...] + s_ref[...]).astype(jnp.bfloat16)
    h2 = jnp.dot(zp, w3_ref[...].astype(jnp.bfloat16),
                 preferred_element_type=jnp.float32)
    h2 = jnp.maximum(h2 + b3_ref[...], 0.0).astype(jnp.bfloat16)
    y = jnp.dot(h2, w4_ref[...].astype(jnp.bfloat16),
                preferred_element_type=jnp.float32)
    y = y + b4_ref[...]
    o_ref[...] = jnp.maximum(y, 0.0) + jnp.log1p(jnp.exp(-jnp.abs(y)))


def _decoder(z, shift, w3, b3, w4, b4, bm):
    B, L = z.shape
    H = w3.shape[1]
    G = w4.shape[1]
    grid = (B // bm,)
    return pl.pallas_call(
        _dec_body,
        grid=grid,
        in_specs=[
            pl.BlockSpec((bm, L), lambda i: (i, 0)),
            pl.BlockSpec((bm, L), lambda i: (i, 0)),
            pl.BlockSpec((L, H), lambda i: (0, 0)),
            pl.BlockSpec((1, H), lambda i: (0, 0)),
            pl.BlockSpec((H, G), lambda i: (0, 0)),
            pl.BlockSpec((1, G), lambda i: (0, 0)),
        ],
        out_specs=pl.BlockSpec((bm, G), lambda i: (i, 0)),
        out_shape=jax.ShapeDtypeStruct((B, G), jnp.float32),
    )(z, shift, w3, b3, w4, b4)


def kernel(ctrl_expr, pert_idx, W_enc1, b_enc1, W_enc2, b_enc2, pert_emb, W_dec1, b_dec1, W_dec2, b_dec2):
    B = ctrl_expr.shape[0]
    L = pert_emb.shape[1]
    shift = _sc_gather(pert_emb, pert_idx.astype(jnp.int32), B, L)
    z = _encoder(ctrl_expr, W_enc1, b_enc1.reshape(1, -1), W_enc2,
                 b_enc2.reshape(1, -1), bm=512)
    out = _decoder(z, shift, W_dec1, b_dec1.reshape(1, -1), W_dec2,
                   b_dec2.reshape(1, -1), bm=512)
    return out


# speedup vs baseline: 2.0025x; 2.0025x over previous
import functools

import jax
import jax.numpy as jnp
from jax import lax
from jax.experimental import pallas as pl
from jax.experimental.pallas import tpu as pltpu
from jax.experimental.pallas import tpu_sc as plsc


def _sc_gather(table, idx, B, D):
    NW = 32
    b_per_w = B // NW
    mesh = plsc.VectorSubcoreMesh(core_axis_name="c", subcore_axis_name="s")

    @functools.partial(
        pl.kernel,
        mesh=mesh,
        out_type=jax.ShapeDtypeStruct((B, D), jnp.float32),
        scratch_types=[
            pltpu.VMEM((b_per_w,), jnp.int32),
            pltpu.VMEM((b_per_w, D), jnp.float32),
            pltpu.SemaphoreType.DMA,
        ],
    )
    def gather_kernel(table_hbm, idx_hbm, out_hbm, idx_v, rows_v, sem):
        wid = lax.axis_index("s") * 2 + lax.axis_index("c")
        base = wid * b_per_w
        pltpu.sync_copy(idx_hbm.at[pl.ds(base, b_per_w)], idx_v)
        pltpu.async_copy(table_hbm.at[idx_v], rows_v, sem).wait()
        pltpu.sync_copy(rows_v, out_hbm.at[pl.ds(base, b_per_w)])

    return gather_kernel(table, idx)


def _enc_body(ct_ref, w1t_ref, b1_ref, w2t_ref, b2_ref, zt_ref):
    ct = ct_ref[...].astype(jnp.bfloat16)
    w1t = w1t_ref[...].astype(jnp.bfloat16)
    ht = jnp.dot(w1t, ct, preferred_element_type=jnp.float32)
    ht = jnp.maximum(ht + b1_ref[...], 0.0).astype(jnp.bfloat16)
    zt = jnp.dot(w2t_ref[...].astype(jnp.bfloat16), ht,
                 preferred_element_type=jnp.float32)
    zt_ref[...] = zt + b2_ref[...]


def _encoder(ct, w1t, b1c, w2t, b2c, bn):
    G, B = ct.shape
    H = w1t.shape[0]
    L = w2t.shape[0]
    grid = (B // bn,)
    return pl.pallas_call(
        _enc_body,
        grid=grid,
        in_specs=[
            pl.BlockSpec((G, bn), lambda i: (0, i)),
            pl.BlockSpec((H, G), lambda i: (0, 0)),
            pl.BlockSpec((H, 1), lambda i: (0, 0)),
            pl.BlockSpec((L, H), lambda i: (0, 0)),
            pl.BlockSpec((L, 1), lambda i: (0, 0)),
        ],
        out_specs=pl.BlockSpec((L, bn), lambda i: (0, i)),
        out_shape=jax.ShapeDtypeStruct((L, B), jnp.float32),
    )(ct, w1t, b1c, w2t, b2c)


def _dec_body(zt_ref, st_ref, w3t_ref, b3_ref, w4t_ref, b4_ref, ot_ref):
    zpt = (zt_ref[...] + st_ref[...]).astype(jnp.bfloat16)
    h2t = jnp.dot(w3t_ref[...].astype(jnp.bfloat16), zpt,
                  preferred_element_type=jnp.float32)
    h2t = jnp.maximum(h2t + b3_ref[...], 0.0).astype(jnp.bfloat16)
    yt = jnp.dot(w4t_ref[...].astype(jnp.bfloat16), h2t,
                 preferred_element_type=jnp.float32)
    yt = yt + b4_ref[...]
    ot_ref[...] = jnp.maximum(yt, 0.0) + jnp.log1p(jnp.exp(-jnp.abs(yt)))


def _decoder(zt, st, w3t, b3c, w4t, b4c, bn):
    L, B = zt.shape
    H = w3t.shape[0]
    G = w4t.shape[0]
    grid = (B // bn,)
    return pl.pallas_call(
        _dec_body,
        grid=grid,
        in_specs=[
            pl.BlockSpec((L, bn), lambda i: (0, i)),
            pl.BlockSpec((L, bn), lambda i: (0, i)),
            pl.BlockSpec((H, L), lambda i: (0, 0)),
            pl.BlockSpec((H, 1), lambda i: (0, 0)),
            pl.BlockSpec((G, H), lambda i: (0, 0)),
            pl.BlockSpec((G, 1), lambda i: (0, 0)),
        ],
        out_specs=pl.BlockSpec((G, bn), lambda i: (0, i)),
        out_shape=jax.ShapeDtypeStruct((G, B), jnp.float32),
    )(zt, st, w3t, b3c, w4t, b4c)


def kernel(ctrl_expr, pert_idx, W_enc1, b_enc1, W_enc2, b_enc2, pert_emb, W_dec1, b_dec1, W_dec2, b_dec2):
    B = ctrl_expr.shape[0]
    L = pert_emb.shape[1]
    shift = _sc_gather(pert_emb, pert_idx.astype(jnp.int32), B, L)
    ct = ctrl_expr.T
    w4t = W_dec2.T
    zt = _encoder(ct, W_enc1.T, b_enc1.reshape(-1, 1), W_enc2.T,
                  b_enc2.reshape(-1, 1), bn=512)
    out_t = _decoder(zt, shift.T, W_dec1.T, b_dec1.reshape(-1, 1), w4t,
                     b_dec2.reshape(-1, 1), bn=512)
    return out_t.T
